# Initial kernel scaffold; baseline (speedup 1.0000x reference)
import jax
import jax.numpy as jnp
from jax.experimental import pallas as pl


def kernel(query_xyz, support_xyz, features):
    raise NotImplementedError("write your pallas kernel here")


# speedup vs baseline: 230.9397x; 230.9397x over previous
import functools

import jax
import jax.numpy as jnp
from jax import lax
from jax.experimental import pallas as pl
from jax.experimental.pallas import tpu as pltpu
from jax.experimental.pallas import tpu_sc as plsc

B = 4
Q = 2048
N = 8192
C = 256
S = 32
R2 = float(0.1 * 0.1)

NC = 2
NS = 16
NW = NC * NS

QPW = (B * Q) // NW
WPB = Q // QPW
CPW = C // NW
NCHUNK = N // 16

_mesh = plsc.VectorSubcoreMesh(core_axis_name="c", subcore_axis_name="s")


def _ballquery_body(sup_hbm, q_hbm, idx_hbm, gxyz_hbm,
                    supp_v, q_v, selbuf, idxout, xyzout):
    wid = lax.axis_index("s") * NC + lax.axis_index("c")
    b = wid // WPB
    qs = (wid % WPB) * QPW

    pltpu.sync_copy(sup_hbm.at[pl.ds(b * 3 * N, 3 * N)], supp_v)
    pltpu.sync_copy(q_hbm.at[pl.ds((b * Q + qs) * 3, QPW * 3)], q_v)

    iota = jnp.arange(16, dtype=jnp.int32)
    zeros16 = jnp.zeros((16,), jnp.int32)

    def per_query(qq, carry):
        base3 = qq * 3
        qx = plsc.load_gather(q_v, [jnp.full((16,), base3, jnp.int32)])
        qy = plsc.load_gather(q_v, [jnp.full((16,), base3 + 1, jnp.int32)])
        qz = plsc.load_gather(q_v, [jnp.full((16,), base3 + 2, jnp.int32)])

        selbuf[pl.ds(0, 16)] = zeros16

        def cond(cr):
            j, cnt = cr
            return jnp.logical_and(j < NCHUNK, jnp.any(cnt < S))

        def body(cr):
            j, cnt = cr
            xs = supp_v[pl.ds(j * 16, 16)]
            ys = supp_v[pl.ds(N + j * 16, 16)]
            zs = supp_v[pl.ds(2 * N + j * 16, 16)]
            dx = xs - qx
            dy = ys - qy
            dz = zs - qz
            d2 = (dx * dx + dy * dy) + dz * dz
            m = d2 < R2
            pre = plsc.cumsum(m.astype(jnp.int32))
            slot = cnt + pre - 1
            idxv = jnp.full((16,), j * 16, jnp.int32) + iota
            plsc.store_scatter(selbuf, [slot], idxv, mask=m)
            cnt2 = cnt + plsc.all_reduce_population_count(m)
            return (j + jnp.int32(1), cnt2)

        _, cnt = lax.while_loop(cond, body, (jnp.int32(0), zeros16))

        b0 = selbuf[pl.ds(0, 16)]
        b1 = selbuf[pl.ds(16, 16)]
        first = jnp.full((16,), jnp.max(jnp.where(iota < 1, b0, 0)), jnp.int32)
        id0 = jnp.where(iota < cnt, b0, first)
        id1 = jnp.where((iota + 16) < cnt, b1, first)

        idxout[pl.ds(qq * S, 16)] = id0
        idxout[pl.ds(qq * S + 16, 16)] = id1

        gx0 = plsc.load_gather(supp_v, [id0]) - qx
        gx1 = plsc.load_gather(supp_v, [id1]) - qx
        gy0 = plsc.load_gather(supp_v, [id0 + N]) - qy
        gy1 = plsc.load_gather(supp_v, [id1 + N]) - qy
        gz0 = plsc.load_gather(supp_v, [id0 + 2 * N]) - qz
        gz1 = plsc.load_gather(supp_v, [id1 + 2 * N]) - qz

        xyzout[pl.ds(qq * S, 16)] = gx0
        xyzout[pl.ds(qq * S + 16, 16)] = gx1
        xyzout[pl.ds(QPW * S + qq * S, 16)] = gy0
        xyzout[pl.ds(QPW * S + qq * S + 16, 16)] = gy1
        xyzout[pl.ds(2 * QPW * S + qq * S, 16)] = gz0
        xyzout[pl.ds(2 * QPW * S + qq * S + 16, 16)] = gz1
        return carry

    lax.fori_loop(0, QPW, per_query, jnp.int32(0))

    pltpu.sync_copy(idxout, idx_hbm.at[pl.ds((b * Q + qs) * S, QPW * S)])
    for d in range(3):
        pltpu.sync_copy(
            xyzout.at[pl.ds(d * QPW * S, QPW * S)],
            gxyz_hbm.at[pl.ds(((b * 3 + d) * Q + qs) * S, QPW * S)],
        )


_cparams = pltpu.CompilerParams(needs_layout_passes=False)

_ballquery = functools.partial(
    pl.kernel,
    mesh=_mesh,
    compiler_params=_cparams,
    out_type=[
        jax.ShapeDtypeStruct((B * Q * S,), jnp.int32),
        jax.ShapeDtypeStruct((B * 3 * Q * S,), jnp.float32),
    ],
    scratch_types=[
        pltpu.VMEM((3 * N,), jnp.float32),
        pltpu.VMEM((QPW * 3,), jnp.float32),
        pltpu.VMEM((48,), jnp.int32),
        pltpu.VMEM((QPW * S,), jnp.int32),
        pltpu.VMEM((3 * QPW * S,), jnp.float32),
    ],
)(_ballquery_body)


def _group_body(feat_hbm, idx_hbm, out_hbm, idx_v, feat_v, out_v):
    wid = lax.axis_index("s") * NC + lax.axis_index("c")

    def per_batch(b, carry):
        pltpu.sync_copy(idx_hbm.at[pl.ds(b * Q * S, Q * S)], idx_v)

        def per_chan(c, carry2):
            ch = wid * CPW + c
            pltpu.sync_copy(
                feat_hbm.at[pl.ds((b * C + ch) * N, N)], feat_v)

            def per_qchunk(qc, carry3):
                def per_query(q, carry4):
                    off = (qc * 1024 + q) * S
                    i0 = idx_v[pl.ds(off, 16)]
                    i1 = idx_v[pl.ds(off + 16, 16)]
                    out_v[pl.ds(q * S, 16)] = plsc.load_gather(feat_v, [i0])
                    out_v[pl.ds(q * S + 16, 16)] = plsc.load_gather(feat_v, [i1])
                    return carry4

                lax.fori_loop(0, 1024, per_query, jnp.int32(0))
                pltpu.sync_copy(
                    out_v,
                    out_hbm.at[pl.ds(((b * C + ch) * Q + qc * 1024) * S,
                                     1024 * S)],
                )
                return carry3

            lax.fori_loop(0, Q // 1024, per_qchunk, jnp.int32(0))
            return carry2

        lax.fori_loop(0, CPW, per_chan, jnp.int32(0))
        return carry

    lax.fori_loop(0, B, per_batch, jnp.int32(0))


_group = functools.partial(
    pl.kernel,
    mesh=_mesh,
    compiler_params=_cparams,
    out_type=jax.ShapeDtypeStruct((B * C * Q * S,), jnp.float32),
    scratch_types=[
        pltpu.VMEM((Q * S,), jnp.int32),
        pltpu.VMEM((N,), jnp.float32),
        pltpu.VMEM((1024 * S,), jnp.float32),
    ],
)(_group_body)


@jax.jit
def kernel(query_xyz, support_xyz, features):
    sup_t = jnp.transpose(support_xyz, (0, 2, 1)).reshape(-1)
    q_flat = query_xyz.reshape(-1)
    idx_flat, gxyz_flat = _ballquery(sup_t, q_flat)
    gfeat_flat = _group(features.reshape(-1), idx_flat)
    grouped_xyz = gxyz_flat.reshape(B, 3, Q, S)
    grouped_features = gfeat_flat.reshape(B, C, Q, S)
    return grouped_xyz, grouped_features


# speedup vs baseline: 803.4971x; 3.4793x over previous
import functools

import jax
import jax.numpy as jnp
from jax import lax
from jax.experimental import pallas as pl
from jax.experimental.pallas import tpu as pltpu
from jax.experimental.pallas import tpu_sc as plsc

B = 4
Q = 2048
N = 8192
C = 256
S = 32
R2 = float(0.1 * 0.1)

NC = 2
NS = 16
NW = NC * NS

QPW = (B * Q) // NW
WPB = Q // QPW
CPW = C // NW
NCHUNK = N // 16

_mesh = plsc.VectorSubcoreMesh(core_axis_name="c", subcore_axis_name="s")


def _ballquery_body(sup_hbm, q_hbm, idx_hbm, gxyz_hbm,
                    supp_v, q_v, selbuf, idxout, xyzout):
    wid = lax.axis_index("s") * NC + lax.axis_index("c")
    b = wid // WPB
    qs = (wid % WPB) * QPW

    pltpu.sync_copy(sup_hbm.at[pl.ds(b * 3 * N, 3 * N)], supp_v)
    pltpu.sync_copy(q_hbm.at[pl.ds((b * Q + qs) * 3, QPW * 3)], q_v)

    iota = jnp.arange(16, dtype=jnp.int32)
    zeros16 = jnp.zeros((16,), jnp.int32)

    def per_query(qq, carry):
        base3 = qq * 3
        qx = plsc.load_gather(q_v, [jnp.full((16,), base3, jnp.int32)])
        qy = plsc.load_gather(q_v, [jnp.full((16,), base3 + 1, jnp.int32)])
        qz = plsc.load_gather(q_v, [jnp.full((16,), base3 + 2, jnp.int32)])

        selbuf[pl.ds(0, 16)] = zeros16

        @plsc.parallel_loop(0, NCHUNK, unroll=8, carry=zeros16)
        def scan_chunks(j, cnt):
            xs = supp_v[pl.ds(j * 16, 16)]
            ys = supp_v[pl.ds(N + j * 16, 16)]
            zs = supp_v[pl.ds(2 * N + j * 16, 16)]
            dx = xs - qx
            dy = ys - qy
            dz = zs - qz
            d2 = (dx * dx + dy * dy) + dz * dz
            m = d2 < R2
            pre = plsc.cumsum(m.astype(jnp.int32))
            slot = cnt + pre - 1
            idxv = jnp.full((16,), j * 16, jnp.int32) + iota
            wm = jnp.logical_and(m, slot < 48)
            plsc.store_scatter(selbuf, [slot], idxv, mask=wm)
            return cnt + plsc.all_reduce_population_count(m)

        cnt = scan_chunks

        b0 = selbuf[pl.ds(0, 16)]
        b1 = selbuf[pl.ds(16, 16)]
        first = jnp.full((16,), jnp.max(jnp.where(iota < 1, b0, 0)), jnp.int32)
        id0 = jnp.where(iota < cnt, b0, first)
        id1 = jnp.where((iota + 16) < cnt, b1, first)

        idxout[pl.ds(qq * S, 16)] = id0
        idxout[pl.ds(qq * S + 16, 16)] = id1

        gx0 = plsc.load_gather(supp_v, [id0]) - qx
        gx1 = plsc.load_gather(supp_v, [id1]) - qx
        gy0 = plsc.load_gather(supp_v, [id0 + N]) - qy
        gy1 = plsc.load_gather(supp_v, [id1 + N]) - qy
        gz0 = plsc.load_gather(supp_v, [id0 + 2 * N]) - qz
        gz1 = plsc.load_gather(supp_v, [id1 + 2 * N]) - qz

        xyzout[pl.ds(qq * S, 16)] = gx0
        xyzout[pl.ds(qq * S + 16, 16)] = gx1
        xyzout[pl.ds(QPW * S + qq * S, 16)] = gy0
        xyzout[pl.ds(QPW * S + qq * S + 16, 16)] = gy1
        xyzout[pl.ds(2 * QPW * S + qq * S, 16)] = gz0
        xyzout[pl.ds(2 * QPW * S + qq * S + 16, 16)] = gz1
        return carry

    lax.fori_loop(0, QPW, per_query, jnp.int32(0))

    pltpu.sync_copy(idxout, idx_hbm.at[pl.ds((b * Q + qs) * S, QPW * S)])
    for d in range(3):
        pltpu.sync_copy(
            xyzout.at[pl.ds(d * QPW * S, QPW * S)],
            gxyz_hbm.at[pl.ds(((b * 3 + d) * Q + qs) * S, QPW * S)],
        )


_cparams = pltpu.CompilerParams(needs_layout_passes=False)

_ballquery = functools.partial(
    pl.kernel,
    mesh=_mesh,
    compiler_params=_cparams,
    out_type=[
        jax.ShapeDtypeStruct((B * Q * S,), jnp.int32),
        jax.ShapeDtypeStruct((B * 3 * Q * S,), jnp.float32),
    ],
    scratch_types=[
        pltpu.VMEM((3 * N,), jnp.float32),
        pltpu.VMEM((QPW * 3,), jnp.float32),
        pltpu.VMEM((48,), jnp.int32),
        pltpu.VMEM((QPW * S,), jnp.int32),
        pltpu.VMEM((3 * QPW * S,), jnp.float32),
    ],
)(_ballquery_body)


def _group_body(feat_hbm, idx_hbm, out_hbm, idx_v, feat_v, out_v):
    wid = lax.axis_index("s") * NC + lax.axis_index("c")

    def per_batch(b, carry):
        pltpu.sync_copy(idx_hbm.at[pl.ds(b * Q * S, Q * S)], idx_v)

        def per_chan(c, carry2):
            ch = wid * CPW + c
            pltpu.sync_copy(
                feat_hbm.at[pl.ds((b * C + ch) * N, N)], feat_v)

            def per_qchunk(qc, carry3):
                @plsc.parallel_loop(0, 1024, unroll=8)
                def per_query(q):
                    off = (qc * 1024 + q) * S
                    i0 = idx_v[pl.ds(off, 16)]
                    i1 = idx_v[pl.ds(off + 16, 16)]
                    out_v[pl.ds(q * S, 16)] = plsc.load_gather(feat_v, [i0])
                    out_v[pl.ds(q * S + 16, 16)] = plsc.load_gather(feat_v, [i1])

                pltpu.sync_copy(
                    out_v,
                    out_hbm.at[pl.ds(((b * C + ch) * Q + qc * 1024) * S,
                                     1024 * S)],
                )
                return carry3

            lax.fori_loop(0, Q // 1024, per_qchunk, jnp.int32(0))
            return carry2

        lax.fori_loop(0, CPW, per_chan, jnp.int32(0))
        return carry

    lax.fori_loop(0, B, per_batch, jnp.int32(0))


_group = functools.partial(
    pl.kernel,
    mesh=_mesh,
    compiler_params=_cparams,
    out_type=jax.ShapeDtypeStruct((B * C * Q * S,), jnp.float32),
    scratch_types=[
        pltpu.VMEM((Q * S,), jnp.int32),
        pltpu.VMEM((N,), jnp.float32),
        pltpu.VMEM((1024 * S,), jnp.float32),
    ],
)(_group_body)


@jax.jit
def kernel(query_xyz, support_xyz, features):
    sup_t = jnp.transpose(support_xyz, (0, 2, 1)).reshape(-1)
    q_flat = query_xyz.reshape(-1)
    idx_flat, gxyz_flat = _ballquery(sup_t, q_flat)
    gfeat_flat = _group(features.reshape(-1), idx_flat)
    grouped_xyz = gxyz_flat.reshape(B, 3, Q, S)
    grouped_features = gfeat_flat.reshape(B, C, Q, S)
    return grouped_xyz, grouped_features
